# baseline (speedup 1.0000x reference)
import functools

import jax
import jax.numpy as jnp
from jax import lax
from jax.experimental import pallas as pl
from jax.experimental.pallas import tpu as pltpu
from jax.experimental.pallas import tpu_sc as plsc

SAT_EMA_K = 0.999
NROWS, NCLS = 16384, 1000
CPAD = 1024
BLK = 512
GRID = NROWS // BLK
NWORKERS = 32
CHUNK = NROWS // NWORKERS
LANES = 16


def _phase1_body(tau_ref, pt_ref, w_ref, s_ref,
                 mp_ref, idx_ref, nll_ref, thr_ref,
                 colsum_acc, mpsum_acc):
    i = pl.program_id(0)

    @pl.when(i == 0)
    def _init():
        colsum_acc[...] = jnp.zeros_like(colsum_acc)
        mpsum_acc[0] = 0.0

    ones_r = jnp.ones((1, NCLS), jnp.float32)
    w = w_ref[...]
    m = jnp.max(w, axis=0, keepdims=True)
    iota = lax.broadcasted_iota(jnp.int32, (NCLS, BLK), 0)
    idx = jnp.min(jnp.where(w == m, iota, NCLS), axis=0)
    ew = jnp.exp(w - m)
    sumexp = lax.dot_general(
        ones_r, ew, (((1,), (0,)), ((), ())))[0]
    inv = 1.0 / sumexp
    mp = inv
    colsum_acc[...] += lax.dot_general(
        ew, inv.reshape(BLK, 1), (((1,), (0,)), ((), ())))
    mpsum_acc[0] += jnp.sum(mp)

    s = s_ref[...]
    ms = jnp.max(s, axis=0, keepdims=True)
    es = jnp.exp(s - ms)
    ses = lax.dot_general(ones_r, es, (((1,), (0,)), ((), ())))[0]
    lses = ms[0] + jnp.log(ses)
    sval = lax.dot_general(
        ones_r, jnp.where(iota == idx[None, :], s, 0.0),
        (((1,), (0,)), ((), ())))[0]

    mp_ref[...] = mp
    idx_ref[...] = idx
    nll_ref[...] = lses - sval

    @pl.when(i == GRID - 1)
    def _finish():
        colsum_row = jnp.reshape(colsum_acc[...], (1, NCLS))[0]
        p_new = pt_ref[...] * SAT_EMA_K + (1.0 - SAT_EMA_K) * (colsum_row / NROWS)
        tau_new = tau_ref[0] * SAT_EMA_K + (1.0 - SAT_EMA_K) * (mpsum_acc[0] / NROWS)
        thr_ref[pl.ds(0, NCLS)] = p_new * (tau_new / jnp.max(p_new))
        thr_ref[pl.ds(NCLS, CPAD - NCLS)] = jnp.zeros((CPAD - NCLS,), jnp.float32)


def _phase1(wt, st, tau, pt_pad):
    return pl.pallas_call(
        _phase1_body,
        grid=(GRID,),
        in_specs=[
            pl.BlockSpec(memory_space=pltpu.SMEM),
            pl.BlockSpec((NCLS,), lambda i: (0,)),
            pl.BlockSpec((NCLS, BLK), lambda i: (0, i)),
            pl.BlockSpec((NCLS, BLK), lambda i: (0, i)),
        ],
        out_specs=[
            pl.BlockSpec((BLK,), lambda i: (i,)),
            pl.BlockSpec((BLK,), lambda i: (i,)),
            pl.BlockSpec((BLK,), lambda i: (i,)),
            pl.BlockSpec((CPAD,), lambda i: (0,)),
        ],
        out_shape=[
            jax.ShapeDtypeStruct((NROWS,), jnp.float32),
            jax.ShapeDtypeStruct((NROWS,), jnp.int32),
            jax.ShapeDtypeStruct((NROWS,), jnp.float32),
            jax.ShapeDtypeStruct((CPAD,), jnp.float32),
        ],
        scratch_shapes=[
            pltpu.VMEM((NCLS, 1), jnp.float32),
            pltpu.SMEM((1,), jnp.float32),
        ],
    )(tau, pt_pad, wt, st)


def _phase2_sc_body(idx_hbm, mp_hbm, nll_hbm, tbl_hbm,
                    mask_hbm, part_hbm,
                    idx_v, mp_v, nll_v, tbl_v, mask_v, acc_v, sem):
    wid = lax.axis_index("s") * 2 + lax.axis_index("c")
    base = wid * CHUNK
    c1 = pltpu.async_copy(idx_hbm.at[pl.ds(base, CHUNK)], idx_v, sem)
    c2 = pltpu.async_copy(mp_hbm.at[pl.ds(base, CHUNK)], mp_v, sem)
    c3 = pltpu.async_copy(nll_hbm.at[pl.ds(base, CHUNK)], nll_v, sem)
    c4 = pltpu.async_copy(tbl_hbm, tbl_v, sem)
    c1.wait(); c2.wait(); c3.wait(); c4.wait()

    def body(j, acc):
        o = j * LANES
        iv = idx_v[pl.ds(o, LANES)]
        thr = plsc.load_gather(tbl_v, [iv])
        mv = jnp.where(mp_v[pl.ds(o, LANES)] >= thr, 1.0, 0.0)
        mask_v[pl.ds(o, LANES)] = mv
        return acc + nll_v[pl.ds(o, LANES)] * mv

    acc = lax.fori_loop(0, CHUNK // LANES, body,
                        jnp.zeros((LANES,), jnp.float32))
    acc_v[...] = acc * (1.0 / NROWS)
    pltpu.sync_copy(mask_v, mask_hbm.at[pl.ds(base, CHUNK)])
    pltpu.sync_copy(acc_v, part_hbm.at[wid])


@functools.lru_cache(maxsize=1)
def _phase2():
    return pl.kernel(
        _phase2_sc_body,
        out_type=[
            jax.ShapeDtypeStruct((NROWS,), jnp.float32),
            jax.ShapeDtypeStruct((NWORKERS, LANES), jnp.float32),
        ],
        mesh=plsc.VectorSubcoreMesh(core_axis_name="c", subcore_axis_name="s"),
        compiler_params=pltpu.CompilerParams(needs_layout_passes=False),
        scratch_types=[
            pltpu.VMEM((CHUNK,), jnp.int32),
            pltpu.VMEM((CHUNK,), jnp.float32),
            pltpu.VMEM((CHUNK,), jnp.float32),
            pltpu.VMEM((CPAD,), jnp.float32),
            pltpu.VMEM((CHUNK,), jnp.float32),
            pltpu.VMEM((LANES,), jnp.float32),
            pltpu.SemaphoreType.DMA,
        ],
    )


def kernel(logits_ulb_w, logits_ulb_s, tau_t, p_t, label_hist):
    del label_hist
    mp1, idx1, nll1, tbl = _phase1(logits_ulb_w.T, logits_ulb_s.T,
                                   tau_t.reshape(1), p_t)
    mask, parts = _phase2()(idx1, mp1, nll1, tbl)
    loss = jnp.sum(parts)
    return loss, mask


# speedup vs baseline: 1.0763x; 1.0763x over previous
import functools

import jax
import jax.numpy as jnp
from jax import lax
from jax.experimental import pallas as pl
from jax.experimental.pallas import tpu as pltpu
from jax.experimental.pallas import tpu_sc as plsc

SAT_EMA_K = 0.999
NROWS, NCLS = 16384, 1000
CPAD = 1024
BLK = 1024
BLKH = BLK // 2
GRID = NROWS // BLK
NWORKERS = 32
CHUNK = NROWS // NWORKERS
LANES = 16


def _half_stats(w, s, colsum_acc):
    blk = w.shape[1]
    ones_r = jnp.ones((1, NCLS), jnp.float32)
    m = jnp.max(w, axis=0, keepdims=True)
    iota = lax.broadcasted_iota(jnp.int32, (NCLS, blk), 0)
    idx = jnp.min(jnp.where(w == m, iota, NCLS), axis=0)
    ew = jnp.exp(w - m)
    sumexp = lax.dot_general(
        ones_r, ew, (((1,), (0,)), ((), ())))[0]
    inv = 1.0 / sumexp
    mp = inv
    colsum_acc[...] += lax.dot_general(
        ew, inv.reshape(blk, 1), (((1,), (0,)), ((), ())))

    ms = jnp.max(s, axis=0, keepdims=True)
    es = jnp.exp(s - ms)
    ses = lax.dot_general(ones_r, es, (((1,), (0,)), ((), ())))[0]
    lses = ms[0] + jnp.log(ses)
    sval = lax.dot_general(
        ones_r, jnp.where(iota == idx[None, :], s, 0.0),
        (((1,), (0,)), ((), ())))[0]
    return mp, idx, lses - sval


def _phase1_body(tau_ref, pt_ref, wa_ref, sa_ref, wb_ref, sb_ref,
                 mp_ref, idx_ref, nll_ref, thr_ref,
                 colsum_acc, mpsum_acc):
    i = pl.program_id(0)

    @pl.when(i == 0)
    def _init():
        colsum_acc[...] = jnp.zeros_like(colsum_acc)
        mpsum_acc[0] = 0.0

    mp_a, idx_a, nll_a = _half_stats(wa_ref[...], sa_ref[...], colsum_acc)
    mp_b, idx_b, nll_b = _half_stats(wb_ref[...], sb_ref[...], colsum_acc)
    mpsum_acc[0] += jnp.sum(mp_a) + jnp.sum(mp_b)

    mp_ref[pl.ds(0, BLKH)] = mp_a
    mp_ref[pl.ds(BLKH, BLKH)] = mp_b
    idx_ref[pl.ds(0, BLKH)] = idx_a
    idx_ref[pl.ds(BLKH, BLKH)] = idx_b
    nll_ref[pl.ds(0, BLKH)] = nll_a
    nll_ref[pl.ds(BLKH, BLKH)] = nll_b

    @pl.when(i == GRID - 1)
    def _finish():
        colsum_row = jnp.reshape(colsum_acc[...], (1, NCLS))[0]
        p_new = pt_ref[...] * SAT_EMA_K + (1.0 - SAT_EMA_K) * (colsum_row / NROWS)
        tau_new = tau_ref[0] * SAT_EMA_K + (1.0 - SAT_EMA_K) * (mpsum_acc[0] / NROWS)
        thr_ref[pl.ds(0, NCLS)] = p_new * (tau_new / jnp.max(p_new))
        thr_ref[pl.ds(NCLS, CPAD - NCLS)] = jnp.zeros((CPAD - NCLS,), jnp.float32)


def _phase1(wt, st, tau, pt_pad):
    return pl.pallas_call(
        _phase1_body,
        grid=(GRID,),
        in_specs=[
            pl.BlockSpec(memory_space=pltpu.SMEM),
            pl.BlockSpec((NCLS,), lambda i: (0,)),
            pl.BlockSpec((NCLS, BLKH), lambda i: (0, 2 * i)),
            pl.BlockSpec((NCLS, BLKH), lambda i: (0, 2 * i)),
            pl.BlockSpec((NCLS, BLKH), lambda i: (0, 2 * i + 1)),
            pl.BlockSpec((NCLS, BLKH), lambda i: (0, 2 * i + 1)),
        ],
        out_specs=[
            pl.BlockSpec((BLK,), lambda i: (i,)),
            pl.BlockSpec((BLK,), lambda i: (i,)),
            pl.BlockSpec((BLK,), lambda i: (i,)),
            pl.BlockSpec((CPAD,), lambda i: (0,)),
        ],
        out_shape=[
            jax.ShapeDtypeStruct((NROWS,), jnp.float32),
            jax.ShapeDtypeStruct((NROWS,), jnp.int32),
            jax.ShapeDtypeStruct((NROWS,), jnp.float32),
            jax.ShapeDtypeStruct((CPAD,), jnp.float32),
        ],
        scratch_shapes=[
            pltpu.VMEM((NCLS, 1), jnp.float32),
            pltpu.SMEM((1,), jnp.float32),
        ],
    )(tau, pt_pad, wt, st, wt, st)


def _phase2_sc_body(idx_hbm, mp_hbm, nll_hbm, tbl_hbm,
                    mask_hbm, part_hbm,
                    idx_v, mp_v, nll_v, tbl_v, mask_v, acc_v, sem):
    wid = lax.axis_index("s") * 2 + lax.axis_index("c")
    base = wid * CHUNK
    c1 = pltpu.async_copy(idx_hbm.at[pl.ds(base, CHUNK)], idx_v, sem)
    c2 = pltpu.async_copy(mp_hbm.at[pl.ds(base, CHUNK)], mp_v, sem)
    c3 = pltpu.async_copy(nll_hbm.at[pl.ds(base, CHUNK)], nll_v, sem)
    c4 = pltpu.async_copy(tbl_hbm, tbl_v, sem)
    c1.wait(); c2.wait(); c3.wait(); c4.wait()

    def body(j, acc):
        o = j * LANES
        iv = idx_v[pl.ds(o, LANES)]
        thr = plsc.load_gather(tbl_v, [iv])
        mv = jnp.where(mp_v[pl.ds(o, LANES)] >= thr, 1.0, 0.0)
        mask_v[pl.ds(o, LANES)] = mv
        return acc + nll_v[pl.ds(o, LANES)] * mv

    acc = lax.fori_loop(0, CHUNK // LANES, body,
                        jnp.zeros((LANES,), jnp.float32))
    acc_v[...] = acc * (1.0 / NROWS)
    pltpu.sync_copy(mask_v, mask_hbm.at[pl.ds(base, CHUNK)])
    pltpu.sync_copy(acc_v, part_hbm.at[wid])


@functools.lru_cache(maxsize=1)
def _phase2():
    return pl.kernel(
        _phase2_sc_body,
        out_type=[
            jax.ShapeDtypeStruct((NROWS,), jnp.float32),
            jax.ShapeDtypeStruct((NWORKERS, LANES), jnp.float32),
        ],
        mesh=plsc.VectorSubcoreMesh(core_axis_name="c", subcore_axis_name="s"),
        compiler_params=pltpu.CompilerParams(needs_layout_passes=False),
        scratch_types=[
            pltpu.VMEM((CHUNK,), jnp.int32),
            pltpu.VMEM((CHUNK,), jnp.float32),
            pltpu.VMEM((CHUNK,), jnp.float32),
            pltpu.VMEM((CPAD,), jnp.float32),
            pltpu.VMEM((CHUNK,), jnp.float32),
            pltpu.VMEM((LANES,), jnp.float32),
            pltpu.SemaphoreType.DMA,
        ],
    )


def kernel(logits_ulb_w, logits_ulb_s, tau_t, p_t, label_hist):
    del label_hist
    mp1, idx1, nll1, tbl = _phase1(logits_ulb_w.T, logits_ulb_s.T,
                                   tau_t.reshape(1), p_t)
    mask, parts = _phase2()(idx1, mp1, nll1, tbl)
    loss = jnp.sum(parts)
    return loss, mask


# speedup vs baseline: 1.1042x; 1.0259x over previous
import functools

import jax
import jax.numpy as jnp
from jax import lax
from jax.experimental import pallas as pl
from jax.experimental.pallas import tpu as pltpu
from jax.experimental.pallas import tpu_sc as plsc

SAT_EMA_K = 0.999
NROWS, NCLS = 16384, 1000
CPAD = 1024
BLK = 1024
BLKH = BLK // 2
GRID = NROWS // BLK
NWORKERS = 16
CHUNK = NROWS // NWORKERS
LANES = 16


def _half_stats(w, s, colsum_acc):
    blk = w.shape[1]
    ones_r = jnp.ones((1, NCLS), jnp.float32)
    m = jnp.max(w, axis=0, keepdims=True)
    iota = lax.broadcasted_iota(jnp.int32, (NCLS, blk), 0)
    idx = jnp.min(jnp.where(w == m, iota, NCLS), axis=0)
    ew = jnp.exp(w - m)
    sumexp = lax.dot_general(
        ones_r, ew, (((1,), (0,)), ((), ())))[0]
    inv = 1.0 / sumexp
    mp = inv
    colsum_acc[...] += lax.dot_general(
        ew, inv.reshape(blk, 1), (((1,), (0,)), ((), ())))

    ms = jnp.max(s, axis=0, keepdims=True)
    es = jnp.exp(s - ms)
    ses = lax.dot_general(ones_r, es, (((1,), (0,)), ((), ())))[0]
    lses = ms[0] + jnp.log(ses)
    sval = lax.dot_general(
        ones_r, jnp.where(iota == idx[None, :], s, 0.0),
        (((1,), (0,)), ((), ())))[0]
    return mp, idx, lses - sval


def _phase1_body(tau_ref, pt_ref, wa_ref, sa_ref, wb_ref, sb_ref,
                 mp_ref, idx_ref, nll_ref, thr_ref,
                 colsum_acc, mpsum_acc):
    i = pl.program_id(0)

    @pl.when(i == 0)
    def _init():
        colsum_acc[...] = jnp.zeros_like(colsum_acc)
        mpsum_acc[0] = 0.0

    mp_a, idx_a, nll_a = _half_stats(wa_ref[...], sa_ref[...], colsum_acc)
    mp_b, idx_b, nll_b = _half_stats(wb_ref[...], sb_ref[...], colsum_acc)
    mpsum_acc[0] += jnp.sum(mp_a) + jnp.sum(mp_b)

    mp_ref[pl.ds(0, BLKH)] = mp_a
    mp_ref[pl.ds(BLKH, BLKH)] = mp_b
    idx_ref[pl.ds(0, BLKH)] = idx_a
    idx_ref[pl.ds(BLKH, BLKH)] = idx_b
    nll_ref[pl.ds(0, BLKH)] = nll_a
    nll_ref[pl.ds(BLKH, BLKH)] = nll_b

    @pl.when(i == GRID - 1)
    def _finish():
        colsum_row = jnp.reshape(colsum_acc[...], (1, NCLS))[0]
        p_new = pt_ref[...] * SAT_EMA_K + (1.0 - SAT_EMA_K) * (colsum_row / NROWS)
        tau_new = tau_ref[0] * SAT_EMA_K + (1.0 - SAT_EMA_K) * (mpsum_acc[0] / NROWS)
        thr_ref[pl.ds(0, NCLS)] = p_new * (tau_new / jnp.max(p_new))
        thr_ref[pl.ds(NCLS, CPAD - NCLS)] = jnp.zeros((CPAD - NCLS,), jnp.float32)


def _phase1(wt, st, tau, pt_pad):
    return pl.pallas_call(
        _phase1_body,
        grid=(GRID,),
        in_specs=[
            pl.BlockSpec(memory_space=pltpu.SMEM),
            pl.BlockSpec((NCLS,), lambda i: (0,)),
            pl.BlockSpec((NCLS, BLKH), lambda i: (0, 2 * i)),
            pl.BlockSpec((NCLS, BLKH), lambda i: (0, 2 * i)),
            pl.BlockSpec((NCLS, BLKH), lambda i: (0, 2 * i + 1)),
            pl.BlockSpec((NCLS, BLKH), lambda i: (0, 2 * i + 1)),
        ],
        out_specs=[
            pl.BlockSpec((BLK,), lambda i: (i,)),
            pl.BlockSpec((BLK,), lambda i: (i,)),
            pl.BlockSpec((BLK,), lambda i: (i,)),
            pl.BlockSpec((CPAD,), lambda i: (0,)),
        ],
        out_shape=[
            jax.ShapeDtypeStruct((NROWS,), jnp.float32),
            jax.ShapeDtypeStruct((NROWS,), jnp.int32),
            jax.ShapeDtypeStruct((NROWS,), jnp.float32),
            jax.ShapeDtypeStruct((CPAD,), jnp.float32),
        ],
        scratch_shapes=[
            pltpu.VMEM((NCLS, 1), jnp.float32),
            pltpu.SMEM((1,), jnp.float32),
        ],
    )(tau, pt_pad, wt, st, wt, st)


def _phase2_sc_body(idx_hbm, mp_hbm, nll_hbm, tbl_hbm,
                    mask_hbm, part_hbm,
                    idx_v, mp_v, nll_v, tbl_v, mask_v, acc_v, sem):
    wid = lax.axis_index("s")
    base = wid * CHUNK
    c1 = pltpu.async_copy(idx_hbm.at[pl.ds(base, CHUNK)], idx_v, sem)
    c2 = pltpu.async_copy(mp_hbm.at[pl.ds(base, CHUNK)], mp_v, sem)
    c3 = pltpu.async_copy(nll_hbm.at[pl.ds(base, CHUNK)], nll_v, sem)
    c4 = pltpu.async_copy(tbl_hbm, tbl_v, sem)
    c1.wait(); c2.wait(); c3.wait(); c4.wait()

    def body(j, acc):
        o = j * LANES
        iv = idx_v[pl.ds(o, LANES)]
        thr = plsc.load_gather(tbl_v, [iv])
        mv = jnp.where(mp_v[pl.ds(o, LANES)] >= thr, 1.0, 0.0)
        mask_v[pl.ds(o, LANES)] = mv
        return acc + nll_v[pl.ds(o, LANES)] * mv

    acc = lax.fori_loop(0, CHUNK // LANES, body,
                        jnp.zeros((LANES,), jnp.float32))
    acc_v[...] = acc * (1.0 / NROWS)
    pltpu.sync_copy(mask_v, mask_hbm.at[pl.ds(base, CHUNK)])
    pltpu.sync_copy(acc_v, part_hbm.at[wid])


@functools.lru_cache(maxsize=1)
def _phase2():
    return pl.kernel(
        _phase2_sc_body,
        out_type=[
            jax.ShapeDtypeStruct((NROWS,), jnp.float32),
            jax.ShapeDtypeStruct((NWORKERS, LANES), jnp.float32),
        ],
        mesh=plsc.VectorSubcoreMesh(core_axis_name="c", subcore_axis_name="s",
                                    num_cores=1),
        compiler_params=pltpu.CompilerParams(needs_layout_passes=False),
        scratch_types=[
            pltpu.VMEM((CHUNK,), jnp.int32),
            pltpu.VMEM((CHUNK,), jnp.float32),
            pltpu.VMEM((CHUNK,), jnp.float32),
            pltpu.VMEM((CPAD,), jnp.float32),
            pltpu.VMEM((CHUNK,), jnp.float32),
            pltpu.VMEM((LANES,), jnp.float32),
            pltpu.SemaphoreType.DMA,
        ],
    )


def kernel(logits_ulb_w, logits_ulb_s, tau_t, p_t, label_hist):
    del label_hist
    mp1, idx1, nll1, tbl = _phase1(logits_ulb_w.T, logits_ulb_s.T,
                                   tau_t.reshape(1), p_t)
    mask, parts = _phase2()(idx1, mp1, nll1, tbl)
    loss = jnp.sum(parts)
    return loss, mask
